# baseline (speedup 1.0000x reference)
import jax
import jax.numpy as jnp
from jax.experimental import pallas as pl
from jax.experimental.pallas import tpu as pltpu


def _body(emb_ref, anchor_ref, out_ref, sem):
    B, S, D = out_ref.shape
    cps = []
    for b in range(B):
        cp = pltpu.make_async_copy(
            anchor_ref.at[0, :],
            out_ref.at[b, 0, :],
            sem.at[b],
        )
        cp.start()
        cps.append(cp)
    for cp in cps:
        cp.wait()


@jax.jit
def _run(token_embeddings, style_anchor):
    B, S, D = token_embeddings.shape
    return pl.pallas_call(
        _body,
        in_specs=[
            pl.BlockSpec(memory_space=pltpu.MemorySpace.HBM),
            pl.BlockSpec(memory_space=pltpu.MemorySpace.HBM),
        ],
        out_specs=pl.BlockSpec(memory_space=pltpu.MemorySpace.HBM),
        out_shape=jax.ShapeDtypeStruct((B, S, D), token_embeddings.dtype),
        input_output_aliases={0: 0},
        scratch_shapes=[pltpu.SemaphoreType.DMA((B,))],
    )(token_embeddings, style_anchor)


def kernel(token_embeddings, style_anchor):
    return _run(token_embeddings, style_anchor)
